# baseline (speedup 1.0000x reference)
import functools

import jax
import jax.numpy as jnp
from jax import lax
from jax.experimental import pallas as pl
from jax.experimental.pallas import tpu as pltpu
from jax.experimental.pallas import tpu_sc as plsc

VOCAB = 1000
EMBED = 64
LANE = 128
NUM_CORES = 2
NUM_SUBCORES = 16
NUM_WORKERS = NUM_CORES * NUM_SUBCORES

IDX_PER_STREAM = 25
SPAIR_PER_PHASE = 8
ROWS_PER_PHASE = IDX_PER_STREAM * SPAIR_PER_PHASE


def _sc_gather(x_grp, table):
    _, n_spair, _, _ = x_grp.shape
    rows_w = n_spair * IDX_PER_STREAM
    n_phase = n_spair // SPAIR_PER_PHASE
    n_pair = n_phase // 2
    total_rows = NUM_WORKERS * rows_w

    mesh = plsc.VectorSubcoreMesh(
        core_axis_name="c", subcore_axis_name="s",
        num_cores=NUM_CORES, num_subcores=NUM_SUBCORES)

    @functools.partial(
        pl.kernel,
        mesh=mesh,
        out_type=jax.ShapeDtypeStruct((total_rows, LANE), jnp.float32),
        scratch_types=[
            pltpu.VMEM((n_spair, 2, IDX_PER_STREAM), jnp.int32),
            pltpu.VMEM((2, ROWS_PER_PHASE, EMBED), jnp.float32),
            pltpu.VMEM((2, ROWS_PER_PHASE, EMBED), jnp.float32),
            pltpu.SemaphoreType.DMA,
            pltpu.SemaphoreType.DMA,
        ],
        compiler_params=pltpu.CompilerParams(use_tc_tiling_on_sc=False),
    )
    def k(x_hbm, table_hbm, out_hbm, idx_v, buf_a, buf_b, sem_a, sem_b):
        wid = lax.axis_index("s") * NUM_CORES + lax.axis_index("c")
        base_w = wid * rows_w

        pltpu.sync_copy(x_hbm.at[wid], idx_v)

        def fire(phase, buf, sem):
            for q in range(SPAIR_PER_PHASE):
                sp = phase * SPAIR_PER_PHASE + q
                rows = pl.ds(q * IDX_PER_STREAM, IDX_PER_STREAM)
                for half in range(2):
                    pltpu.async_copy(
                        table_hbm.at[idx_v.at[sp, half]],
                        buf.at[half, rows],
                        sem)

        def drain_and_store(phase, buf, sem):
            rows = pl.ds(base_w + phase * ROWS_PER_PHASE, ROWS_PER_PHASE)
            halves = [out_hbm.at[rows, pl.ds(h * EMBED, EMBED)]
                      for h in range(2)]
            for h in range(2):
                pltpu.make_async_copy(halves[h], buf.at[h], sem).wait()
            for h in range(2):
                pltpu.sync_copy(buf.at[h], halves[h])

        fire(0, buf_a, sem_a)

        def pair(i, carry):
            pa = 2 * i
            fire(pa + 1, buf_b, sem_b)
            drain_and_store(pa, buf_a, sem_a)

            @pl.when(i < n_pair - 1)
            def _():
                fire(pa + 2, buf_a, sem_a)

            drain_and_store(pa + 1, buf_b, sem_b)
            return carry

        lax.fori_loop(0, n_pair, pair, 0)

    return k(x_grp, table)


def kernel(x, embedding_matrix):
    batch, hist = x.shape
    total = batch * hist
    rows_w = total // NUM_WORKERS // 2
    n_spair = rows_w // IDX_PER_STREAM
    x_grp = (x.astype(jnp.int32)
             .reshape(NUM_WORKERS, n_spair, IDX_PER_STREAM, 2)
             .transpose(0, 1, 3, 2))
    out128 = _sc_gather(x_grp, embedding_matrix)
    return out128.reshape(batch, hist, EMBED)


# speedup vs baseline: 1.1153x; 1.1153x over previous
import functools

import jax
import jax.numpy as jnp
from jax import lax
from jax.experimental import pallas as pl
from jax.experimental.pallas import tpu as pltpu
from jax.experimental.pallas import tpu_sc as plsc

VOCAB = 1000
EMBED = 64
HIST = 50
NUM_CORES = 2
NUM_SUBCORES = 16
NUM_WORKERS = NUM_CORES * NUM_SUBCORES

B_PER_PHASE = 8
CHUNKS = 4


def _sc_gather(x_grp, table):
    _, b_per_w, _ = x_grp.shape
    n_phase = b_per_w // B_PER_PHASE
    n_pair = n_phase // 2
    batch = NUM_WORKERS * b_per_w

    mesh = plsc.VectorSubcoreMesh(
        core_axis_name="c", subcore_axis_name="s",
        num_cores=NUM_CORES, num_subcores=NUM_SUBCORES)

    @functools.partial(
        pl.kernel,
        mesh=mesh,
        out_type=jax.ShapeDtypeStruct((batch, HIST, EMBED), jnp.float32),
        scratch_types=[
            pltpu.VMEM((b_per_w, HIST), jnp.int32),
            pltpu.VMEM((B_PER_PHASE, HIST, EMBED), jnp.float32),
            pltpu.VMEM((B_PER_PHASE, HIST, EMBED), jnp.float32),
            pltpu.SemaphoreType.DMA,
            pltpu.SemaphoreType.DMA,
        ],
        compiler_params=pltpu.CompilerParams(use_tc_tiling_on_sc=False),
    )
    def k(x_hbm, table_hbm, out_hbm, idx_v, buf_a, buf_b, sem_a, sem_b):
        wid = lax.axis_index("s") * NUM_CORES + lax.axis_index("c")
        base_w = wid * b_per_w

        pltpu.sync_copy(x_hbm.at[wid], idx_v)

        def fire(phase, buf, sem):
            for q in range(B_PER_PHASE):
                pltpu.async_copy(
                    table_hbm.at[idx_v.at[phase * B_PER_PHASE + q]],
                    buf.at[q],
                    sem)

        def drain_and_store(phase, buf, sem):
            out_slice = out_hbm.at[pl.ds(base_w + phase * B_PER_PHASE,
                                         B_PER_PHASE)]
            pltpu.make_async_copy(out_slice, buf, sem).wait()
            pltpu.sync_copy(buf, out_slice)

        fire(0, buf_a, sem_a)

        def pair(i, carry):
            pa = 2 * i
            fire(pa + 1, buf_b, sem_b)
            drain_and_store(pa, buf_a, sem_a)

            @pl.when(i < n_pair - 1)
            def _():
                fire(pa + 2, buf_a, sem_a)

            drain_and_store(pa + 1, buf_b, sem_b)
            return carry

        lax.fori_loop(0, n_pair, pair, 0)

    return k(x_grp, table)


def kernel(x, embedding_matrix):
    batch, hist = x.shape
    bc = batch // CHUNKS
    xi = x.astype(jnp.int32)
    outs = []
    for c in range(CHUNKS):
        x_grp = lax.slice_in_dim(xi, c * bc, (c + 1) * bc).reshape(
            NUM_WORKERS, bc // NUM_WORKERS, hist)
        outs.append(_sc_gather(x_grp, embedding_matrix))
    return lax.concatenate(outs, 0)
